# baseline (speedup 1.0000x reference)
import jax
import jax.numpy as jnp
from jax import lax
from jax.experimental import pallas as pl
from jax.experimental.pallas import tpu as pltpu
from jax.experimental.pallas import tpu_sc as plsc

B = 4
T = 4096
D = 1024
L = 16

NC = 2
NS = 16
NW = NC * NS
ROWS = B * T
TW = T // NW
CT = 16
NCT = TW // CT
NSTEP = NCT * B
NTB = 4
SB = 8
NSB = NSTEP // SB


def _body(idx_hbm, tok_hbm, pos_hbm, out_hbm,
          idx_v, tb0, tb1, tb2, tb3, pb0, pb1,
          sg0, sg1, sg2, sg3, ss0, ss1, ss2, ss3, sp0, sp1):
    cid = lax.axis_index("c")
    sid = lax.axis_index("s")
    wid = sid * NC + cid
    t0 = wid * TW
    pltpu.sync_copy(idx_hbm.at[wid], idx_v)

    tbufs = (tb0, tb1, tb2, tb3)
    pbufs = (pb0, pb1)
    gsems = (sg0, sg1, sg2, sg3)
    ssems = (ss0, ss1, ss2, ss3)
    psems = (sp0, sp1)

    def gather(tc, b, i):
        return pltpu.async_copy(tok_hbm.at[idx_v.at[tc, b]],
                                tbufs[i], gsems[i])

    def pos_load(tc, i):
        off = t0 + jnp.minimum(tc, NCT - 1) * CT
        return pltpu.async_copy(pos_hbm.at[pl.ds(off, CT)], pbufs[i],
                                psems[i])

    def store(tc, b, i):
        row = b * T + t0 + tc * CT
        return pltpu.async_copy(tbufs[i], out_hbm.at[pl.ds(row, CT)],
                                ssems[i])

    def wait_gather(i):
        pltpu.make_async_copy(pos_hbm.at[pl.ds(t0, CT)], tbufs[i],
                              gsems[i]).wait()

    def wait_pos(i):
        pltpu.make_async_copy(pos_hbm.at[pl.ds(t0, CT)], pbufs[i],
                              psems[i]).wait()

    def wait_store(i):
        pltpu.make_async_copy(tbufs[i], out_hbm.at[pl.ds(t0, CT)],
                              ssems[i]).wait()

    gather(0, 0, 0)
    gather(0, 1, 1)
    pos_load(0, 0)
    pos_load(1, 1)
    store(0, 2, 2)
    store(0, 3, 3)

    def superblock(sb, carry):
        tc0 = sb * 2
        for u in range(SB):
            b = u % 4
            half = u // 4
            tc = tc0 + half
            i = u % NTB
            if u == 0:
                wait_pos(0)
            if u == 4:
                wait_pos(1)
                pos_load(tc0 + 2, 0)
            wait_store((u + 2) % NTB)
            tc2 = jnp.minimum(tc0 + (u + 2) // 4, NCT - 1)
            gather(tc2, (u + 2) % 4, (u + 2) % NTB)
            wait_gather(i)

            @plsc.parallel_loop(0, CT * D // L, 1, unroll=16)
            def add_slice(q, buf=tbufs[i], pbuf=pbufs[half]):
                r = q // (D // L)
                col = (q % (D // L)) * L
                plsc.addupdate(buf.at[r, pl.ds(col, L)],
                               pbuf[r, pl.ds(col, L)])

            store(tc, b, i)
        pos_load(tc0 + 3, 1)
        return carry

    lax.fori_loop(0, NSB, superblock, 0)
    for i in (0, 1):
        wait_gather(i)
        wait_pos(i)
    wait_store(2)
    wait_store(3)


@jax.jit
def _embed(idx4, token_table, pos_table):
    mesh = plsc.VectorSubcoreMesh(core_axis_name="c", subcore_axis_name="s")
    f = pl.kernel(
        _body,
        out_type=jax.ShapeDtypeStruct((ROWS, D), jnp.float32),
        mesh=mesh,
        scratch_types=[
            pltpu.VMEM((NCT, B, CT), jnp.int32),
            pltpu.VMEM((CT, D), jnp.float32),
            pltpu.VMEM((CT, D), jnp.float32),
            pltpu.VMEM((CT, D), jnp.float32),
            pltpu.VMEM((CT, D), jnp.float32),
            pltpu.VMEM((CT, D), jnp.float32),
            pltpu.VMEM((CT, D), jnp.float32),
        ] + [pltpu.SemaphoreType.DMA] * 10,
    )
    return f(idx4, token_table, pos_table)


def kernel(idx, token_table, pos_table):
    idx4 = idx.reshape(B, NW, NCT, CT).transpose(1, 2, 0, 3)
    out = _embed(idx4, token_table, pos_table)
    return out.reshape(B, T, D)


# speedup vs baseline: 1.0096x; 1.0096x over previous
import jax
import jax.numpy as jnp
from jax import lax
from jax.experimental import pallas as pl
from jax.experimental.pallas import tpu as pltpu
from jax.experimental.pallas import tpu_sc as plsc

B = 4
T = 4096
D = 1024
L = 16

NC = 2
NS = 16
NW = NC * NS
ROWS = B * T
TW = T // NW
CT = 16
NCT = TW // CT
NSTEP = NCT * B
NTB = 4
SB = 8
NSB = NSTEP // SB


def _body(idx_hbm, tok_hbm, pos_hbm, out_hbm,
          idx_v, tb0, tb1, tb2, tb3, pb0, pb1,
          sg0, sg1, sg2, sg3, ss0, ss1, ss2, ss3, sp0, sp1):
    cid = lax.axis_index("c")
    sid = lax.axis_index("s")
    wid = sid * NC + cid
    t0 = wid * TW
    pltpu.sync_copy(idx_hbm.at[wid], idx_v)

    tbufs = (tb0, tb1, tb2, tb3)
    pbufs = (pb0, pb1)
    gsems = (sg0, sg1, sg2, sg3)
    ssems = (ss0, ss1, ss2, ss3)
    psems = (sp0, sp1)

    def gather(tc, b, i):
        return pltpu.async_copy(tok_hbm.at[idx_v.at[tc, b]],
                                tbufs[i], gsems[i])

    def pos_load(tc, i):
        off = t0 + jnp.minimum(tc, NCT - 1) * CT
        return pltpu.async_copy(pos_hbm.at[pl.ds(off, CT)], pbufs[i],
                                psems[i])

    def store(tc, b, i):
        row = b * T + t0 + tc * CT
        return pltpu.async_copy(tbufs[i], out_hbm.at[pl.ds(row, CT)],
                                ssems[i])

    def wait_gather(i):
        pltpu.make_async_copy(pos_hbm.at[pl.ds(t0, CT)], tbufs[i],
                              gsems[i]).wait()

    def wait_pos(i):
        pltpu.make_async_copy(pos_hbm.at[pl.ds(t0, CT)], pbufs[i],
                              psems[i]).wait()

    def wait_store(i):
        pltpu.make_async_copy(tbufs[i], out_hbm.at[pl.ds(t0, CT)],
                              ssems[i]).wait()

    gather(0, 0, 0)
    gather(0, 1, 1)
    pos_load(0, 0)
    pos_load(1, 1)
    store(0, 2, 2)
    store(0, 3, 3)

    def superblock(sb, carry):
        tc0 = sb * 2
        for u in range(SB):
            b = u % 4
            half = u // 4
            tc = tc0 + half
            i = u % NTB
            if u == 0:
                wait_pos(0)
            if u == 4:
                wait_pos(1)
                pos_load(tc0 + 2, 0)
            wait_store((u + 2) % NTB)
            tc2 = jnp.minimum(tc0 + (u + 2) // 4, NCT - 1)
            gather(tc2, (u + 2) % 4, (u + 2) % NTB)
            wait_gather(i)

            @plsc.parallel_loop(0, CT * D // L, 1, unroll=8)
            def add_slice(q, buf=tbufs[i], pbuf=pbufs[half]):
                r = q // (D // L)
                col = (q % (D // L)) * L
                plsc.addupdate(buf.at[r, pl.ds(col, L)],
                               pbuf[r, pl.ds(col, L)])

            store(tc, b, i)
        pos_load(tc0 + 3, 1)
        return carry

    lax.fori_loop(0, NSB, superblock, 0)
    for i in (0, 1):
        wait_gather(i)
        wait_pos(i)
    wait_store(2)
    wait_store(3)


@jax.jit
def _embed(idx4, token_table, pos_table):
    mesh = plsc.VectorSubcoreMesh(core_axis_name="c", subcore_axis_name="s")
    f = pl.kernel(
        _body,
        out_type=jax.ShapeDtypeStruct((ROWS, D), jnp.float32),
        mesh=mesh,
        scratch_types=[
            pltpu.VMEM((NCT, B, CT), jnp.int32),
            pltpu.VMEM((CT, D), jnp.float32),
            pltpu.VMEM((CT, D), jnp.float32),
            pltpu.VMEM((CT, D), jnp.float32),
            pltpu.VMEM((CT, D), jnp.float32),
            pltpu.VMEM((CT, D), jnp.float32),
            pltpu.VMEM((CT, D), jnp.float32),
        ] + [pltpu.SemaphoreType.DMA] * 10,
    )
    return f(idx4, token_table, pos_table)


def kernel(idx, token_table, pos_table):
    idx4 = idx.reshape(B, NW, NCT, CT).transpose(1, 2, 0, 3)
    out = _embed(idx4, token_table, pos_table)
    return out.reshape(B, T, D)
